# baseline (speedup 1.0000x reference)
import jax
import jax.numpy as jnp
from jax import lax
from jax.experimental import pallas as pl
from jax.experimental.pallas import tpu as pltpu
from jax.experimental.pallas import tpu_sc as plsc

_NC = 2
_NS = 16
_NW = _NC * _NS
_D = 64
_DP = 128
_V = 1000000
_TB = _V // _DP
_TREM = _V - _TB * _DP
_TPW = _TB // _NW
_TXTRA = _TB - _TPW * _NW
_CHUNK = 256


def _transpose_block(in_v, out_flat):
    iota_p = lax.iota(jnp.int32, 16) * _DP

    for tg in range(_DP // 16):
        t0 = tg * 16
        base = iota_p + t0 * _DP
        for f in range(_D):
            v = in_v[f, pl.ds(t0, 16)]
            plsc.store_scatter(out_flat, [base + f], v)


_BLK = _DP * _DP


def _relayout_body(wt_hbm, wtail_hbm, wpad_hbm, in0, in1, out0, out1, tail_v,
                   gs0, gs1, ws0, ws1):
    wid = lax.axis_index("s") * _NC + lax.axis_index("c")
    base = wid * _TPW

    def fire_in(c, in_v, gsem):
        pltpu.async_copy(
            wt_hbm.at[:, pl.ds(c * _DP, _DP)], in_v, gsem)

    def wait_in(in_v, gsem):
        pltpu.make_async_copy(
            wt_hbm.at[:, pl.ds(0, _DP)], in_v, gsem).wait()

    def fire_out(c, out_v, wsem):
        pltpu.async_copy(
            out_v, wpad_hbm.at[pl.ds(c * _BLK, _BLK)], wsem)

    def wait_out(out_v, wsem):
        pltpu.make_async_copy(
            out_v, wpad_hbm.at[pl.ds(0, _BLK)], wsem).wait()

    fire_in(base, in0, gs0)

    def pair(p, carry):
        c0 = base + 2 * p
        wait_in(in0, gs0)
        fire_in(c0 + 1, in1, gs1)

        @pl.when(p >= 1)
        def _():
            wait_out(out0, ws0)
        _transpose_block(in0, out0)
        fire_out(c0, out0, ws0)
        wait_in(in1, gs1)

        @pl.when(p * 2 + 2 < _TPW)
        def _():
            fire_in(c0 + 2, in0, gs0)

        @pl.when(p >= 1)
        def _():
            wait_out(out1, ws1)
        _transpose_block(in1, out1)
        fire_out(c0 + 1, out1, ws1)
        return carry

    lax.fori_loop(0, _TPW // 2, pair, 0)
    wait_out(out0, ws0)
    wait_out(out1, ws1)

    @pl.when(wid < _TXTRA)
    def _():
        c = _TB - _TXTRA + wid
        pltpu.sync_copy(wt_hbm.at[:, pl.ds(c * _DP, _DP)], in0)
        _transpose_block(in0, out0)
        pltpu.sync_copy(out0, wpad_hbm.at[pl.ds(c * _BLK, _BLK)])

    @pl.when(wid == _NW - 1)
    def _():
        pltpu.sync_copy(wtail_hbm, tail_v)
        pltpu.sync_copy(tail_v, wpad_hbm.at[pl.ds(_TB * _BLK, _TREM * _DP)])


def _lookup_body(idx_hbm, table_hbm, out_hbm, idx_v, rows0, rows1, gs0, gs1, ws0, ws1):
    wid = lax.axis_index("s") * _NC + lax.axis_index("c")
    n_per_w = idx_v.shape[0]
    n_groups = n_per_w // _CHUNK
    base = wid * n_per_w
    pltpu.sync_copy(idx_hbm.at[pl.ds(base, n_per_w)], idx_v)

    def fire(g, rows, gsem):
        pltpu.async_copy(
            table_hbm.at[idx_v.at[pl.ds(g * _CHUNK, _CHUNK)]], rows, gsem)

    def drain_gather(rows, gsem):
        pltpu.make_async_copy(
            table_hbm.at[idx_v.at[pl.ds(0, _CHUNK)]], rows, gsem).wait()

    def write(g, rows, wsem):
        pltpu.async_copy(
            rows, out_hbm.at[pl.ds(base + g * _CHUNK, _CHUNK)], wsem)

    def wait_write(rows, wsem):
        pltpu.make_async_copy(rows, out_hbm.at[pl.ds(base, _CHUNK)], wsem).wait()

    fire(0, rows0, gs0)
    last = n_groups - 1

    def pair(p, carry):
        r_odd = 2 * p + 1

        @pl.when(p >= 1)
        def _():
            wait_write(rows1, ws1)
        fire(r_odd, rows1, gs1)
        drain_gather(rows0, gs0)
        write(r_odd - 1, rows0, ws0)

        wait_write(rows0, ws0)
        fire(r_odd + 1, rows0, gs0)
        drain_gather(rows1, gs1)
        write(r_odd, rows1, ws1)
        return carry

    lax.fori_loop(0, (n_groups - 2) // 2, pair, 0)
    wait_write(rows1, ws1)
    fire(last, rows1, gs1)
    drain_gather(rows0, gs0)
    write(last - 1, rows0, ws0)
    drain_gather(rows1, gs1)
    write(last, rows1, ws1)
    wait_write(rows0, ws0)
    wait_write(rows1, ws1)


def kernel(tokens, weight):
    s0, s1 = tokens.shape
    b = s0 * s1
    idx = tokens.reshape(b).astype(jnp.int32)
    mesh = plsc.VectorSubcoreMesh(core_axis_name="c", subcore_axis_name="s")

    wtail = jnp.pad(weight[_TB * _DP:], ((0, 0), (0, _DP - _D))).reshape(
        _TREM * _DP)
    wpad1d = pl.kernel(
        _relayout_body,
        out_type=jax.ShapeDtypeStruct((_V * _DP,), jnp.float32),
        mesh=mesh,
        compiler_params=pltpu.CompilerParams(
            use_tc_tiling_on_sc=True, needs_layout_passes=False),
        scratch_types=[
            pltpu.VMEM((_D, _DP), jnp.float32),
            pltpu.VMEM((_D, _DP), jnp.float32),
            pltpu.VMEM((_BLK,), jnp.float32),
            pltpu.VMEM((_BLK,), jnp.float32),
            pltpu.VMEM((_TREM * _DP,), jnp.float32),
            pltpu.SemaphoreType.DMA,
            pltpu.SemaphoreType.DMA,
            pltpu.SemaphoreType.DMA,
            pltpu.SemaphoreType.DMA,
        ],
    )(weight.T, wtail)
    wpad = wpad1d.reshape(_V, _DP)

    out = pl.kernel(
        _lookup_body,
        out_type=jax.ShapeDtypeStruct((b, _DP), jnp.float32),
        mesh=mesh,
        compiler_params=pltpu.CompilerParams(use_tc_tiling_on_sc=False),
        scratch_types=[
            pltpu.VMEM((b // _NW,), jnp.int32),
            pltpu.VMEM((_CHUNK, _DP), jnp.float32),
            pltpu.VMEM((_CHUNK, _DP), jnp.float32),
            pltpu.SemaphoreType.DMA,
            pltpu.SemaphoreType.DMA,
            pltpu.SemaphoreType.DMA,
            pltpu.SemaphoreType.DMA,
        ],
    )(idx, wpad)
    return out[:, :_D].reshape(s0, s1, _D)


# speedup vs baseline: 1.1373x; 1.1373x over previous
import jax
import jax.numpy as jnp
from jax import lax
from jax.experimental import pallas as pl
from jax.experimental.pallas import tpu as pltpu
from jax.experimental.pallas import tpu_sc as plsc

_NC = 2
_NS = 16
_NW = _NC * _NS
_D = 64
_DP = 128
_V = 1000000
_TB = _V // _DP
_TREM = _V - _TB * _DP
_TPW = _TB // _NW
_TXTRA = _TB - _TPW * _NW
_CHUNK = 256


def _transpose_block(in_v, out_v):
    iota = lax.iota(jnp.int32, 16)

    @plsc.parallel_loop(0, _DP // 16, unroll=2)
    def _tgrp(tg):
        t0 = tg * 16
        trow = iota + t0
        for f in range(_D):
            v = in_v[f, pl.ds(t0, 16)]
            fcol = jnp.full((16,), f, jnp.int32)
            plsc.store_scatter(out_v, [trow, fcol], v)


_BLK = _DP * _DP


def _relayout_body(wt_hbm, wtail_hbm, wpad_hbm, in0, in1, out0, out1, tail_v,
                   gs0, gs1, ws0, ws1):
    wid = lax.axis_index("s") * _NC + lax.axis_index("c")
    base = wid * _TPW

    def fire_in(c, in_v, gsem):
        pltpu.async_copy(
            wt_hbm.at[:, pl.ds(c * _DP, _DP)], in_v, gsem)

    def wait_in(in_v, gsem):
        pltpu.make_async_copy(
            wt_hbm.at[:, pl.ds(0, _DP)], in_v, gsem).wait()

    def fire_out(c, out_v, wsem):
        pltpu.async_copy(
            out_v.at[:, pl.ds(0, _DP)], wpad_hbm.at[pl.ds(c * _DP, _DP)], wsem)

    def wait_out(out_v, wsem):
        pltpu.make_async_copy(
            out_v.at[:, pl.ds(0, _DP)], wpad_hbm.at[pl.ds(0, _DP)], wsem).wait()

    fire_in(base, in0, gs0)

    def pair(p, carry):
        c0 = base + 2 * p
        wait_in(in0, gs0)
        fire_in(c0 + 1, in1, gs1)

        @pl.when(p >= 1)
        def _():
            wait_out(out0, ws0)
        _transpose_block(in0, out0)
        fire_out(c0, out0, ws0)
        wait_in(in1, gs1)

        @pl.when(p * 2 + 2 < _TPW)
        def _():
            fire_in(c0 + 2, in0, gs0)

        @pl.when(p >= 1)
        def _():
            wait_out(out1, ws1)
        _transpose_block(in1, out1)
        fire_out(c0 + 1, out1, ws1)
        return carry

    lax.fori_loop(0, _TPW // 2, pair, 0)
    wait_out(out0, ws0)
    wait_out(out1, ws1)

    @pl.when(wid < _TXTRA)
    def _():
        c = _TB - _TXTRA + wid
        pltpu.sync_copy(wt_hbm.at[:, pl.ds(c * _DP, _DP)], in0)
        _transpose_block(in0, out0)
        pltpu.sync_copy(out0.at[:, pl.ds(0, _DP)], wpad_hbm.at[pl.ds(c * _DP, _DP)])

    @pl.when(wid == _NW - 1)
    def _():
        pltpu.sync_copy(wtail_hbm, tail_v)
        pltpu.sync_copy(tail_v, wpad_hbm.at[pl.ds(_TB * _DP, _TREM)])


def _lookup_body(idx_hbm, table_hbm, out_hbm, idx_v, rows0, rows1, gs0, gs1, ws0, ws1):
    wid = lax.axis_index("s") * _NC + lax.axis_index("c")
    n_per_w = idx_v.shape[0]
    n_groups = n_per_w // _CHUNK
    base = wid * n_per_w
    pltpu.sync_copy(idx_hbm.at[pl.ds(base, n_per_w)], idx_v)

    def fire(g, rows, gsem):
        pltpu.async_copy(
            table_hbm.at[idx_v.at[pl.ds(g * _CHUNK, _CHUNK)]], rows, gsem)

    def drain_gather(rows, gsem):
        pltpu.make_async_copy(
            table_hbm.at[idx_v.at[pl.ds(0, _CHUNK)]], rows, gsem).wait()

    def write(g, rows, wsem):
        pltpu.async_copy(
            rows, out_hbm.at[pl.ds(base + g * _CHUNK, _CHUNK)], wsem)

    def wait_write(rows, wsem):
        pltpu.make_async_copy(rows, out_hbm.at[pl.ds(base, _CHUNK)], wsem).wait()

    fire(0, rows0, gs0)
    last = n_groups - 1

    def pair(p, carry):
        r_odd = 2 * p + 1

        @pl.when(p >= 1)
        def _():
            wait_write(rows1, ws1)
        fire(r_odd, rows1, gs1)
        drain_gather(rows0, gs0)
        write(r_odd - 1, rows0, ws0)

        wait_write(rows0, ws0)
        fire(r_odd + 1, rows0, gs0)
        drain_gather(rows1, gs1)
        write(r_odd, rows1, ws1)
        return carry

    lax.fori_loop(0, (n_groups - 2) // 2, pair, 0)
    wait_write(rows1, ws1)
    fire(last, rows1, gs1)
    drain_gather(rows0, gs0)
    write(last - 1, rows0, ws0)
    drain_gather(rows1, gs1)
    write(last, rows1, ws1)
    wait_write(rows0, ws0)
    wait_write(rows1, ws1)


def kernel(tokens, weight):
    s0, s1 = tokens.shape
    b = s0 * s1
    idx = tokens.reshape(b).astype(jnp.int32)
    mesh = plsc.VectorSubcoreMesh(core_axis_name="c", subcore_axis_name="s")

    wtail = jnp.pad(weight[_TB * _DP:], ((0, 0), (0, _DP - _D)))
    wpad = pl.kernel(
        _relayout_body,
        out_type=jax.ShapeDtypeStruct((_V, _DP), jnp.float32),
        mesh=mesh,
        compiler_params=pltpu.CompilerParams(
            use_tc_tiling_on_sc=True, needs_layout_passes=False),
        scratch_types=[
            pltpu.VMEM((_D, _DP), jnp.float32),
            pltpu.VMEM((_D, _DP), jnp.float32),
            pltpu.VMEM((_DP, _DP + 1), jnp.float32),
            pltpu.VMEM((_DP, _DP + 1), jnp.float32),
            pltpu.VMEM((_TREM, _DP), jnp.float32),
            pltpu.SemaphoreType.DMA,
            pltpu.SemaphoreType.DMA,
            pltpu.SemaphoreType.DMA,
            pltpu.SemaphoreType.DMA,
        ],
    )(weight.T, wtail)

    out = pl.kernel(
        _lookup_body,
        out_type=jax.ShapeDtypeStruct((b, _DP), jnp.float32),
        mesh=mesh,
        compiler_params=pltpu.CompilerParams(use_tc_tiling_on_sc=False),
        scratch_types=[
            pltpu.VMEM((b // _NW,), jnp.int32),
            pltpu.VMEM((_CHUNK, _DP), jnp.float32),
            pltpu.VMEM((_CHUNK, _DP), jnp.float32),
            pltpu.SemaphoreType.DMA,
            pltpu.SemaphoreType.DMA,
            pltpu.SemaphoreType.DMA,
            pltpu.SemaphoreType.DMA,
        ],
    )(idx, wpad)
    return out[:, :_D].reshape(s0, s1, _D)


# speedup vs baseline: 1.8227x; 1.6027x over previous
import jax
import jax.numpy as jnp
from jax import lax
from jax.experimental import pallas as pl
from jax.experimental.pallas import tpu as pltpu
from jax.experimental.pallas import tpu_sc as plsc

_NC = 2
_NS = 16
_NW = _NC * _NS
_CHUNK = 512
_D = 64
_DP = 128


def _emb_body(idx_hbm, table_hbm, out_hbm, idx_v, rows0, rows1, gs0, gs1, ws0, ws1):
    wid = lax.axis_index("s") * _NC + lax.axis_index("c")
    n_per_w = idx_v.shape[0]
    n_groups = n_per_w // _CHUNK
    base = wid * n_per_w
    pltpu.sync_copy(idx_hbm.at[pl.ds(base, n_per_w)], idx_v)

    def fire(g, rows, gsem):
        pltpu.async_copy(
            table_hbm.at[idx_v.at[pl.ds(g * _CHUNK, _CHUNK)]], rows, gsem)

    def drain_gather(rows, gsem):
        pltpu.make_async_copy(
            table_hbm.at[idx_v.at[pl.ds(0, _CHUNK)]], rows, gsem).wait()

    def write(g, rows, wsem):
        pltpu.async_copy(
            rows,
            out_hbm.at[pl.ds(base + g * _CHUNK, _CHUNK), pl.ds(0, _D)], wsem)

    def wait_write(rows, wsem):
        pltpu.make_async_copy(
            rows, out_hbm.at[pl.ds(base, _CHUNK), pl.ds(0, _D)], wsem).wait()

    fire(0, rows0, gs0)
    last = n_groups - 1

    def pair(p, carry):
        r_odd = 2 * p + 1

        @pl.when(p >= 1)
        def _():
            wait_write(rows1, ws1)
        fire(r_odd, rows1, gs1)
        drain_gather(rows0, gs0)
        write(r_odd - 1, rows0, ws0)

        wait_write(rows0, ws0)
        fire(r_odd + 1, rows0, gs0)
        drain_gather(rows1, gs1)
        write(r_odd, rows1, ws1)
        return carry

    lax.fori_loop(0, (n_groups - 2) // 2, pair, 0)
    wait_write(rows1, ws1)
    fire(last, rows1, gs1)
    drain_gather(rows0, gs0)
    write(last - 1, rows0, ws0)
    drain_gather(rows1, gs1)
    write(last, rows1, ws1)
    wait_write(rows0, ws0)
    wait_write(rows1, ws1)


def kernel(tokens, weight):
    s0, s1 = tokens.shape
    b = s0 * s1
    idx = tokens.reshape(b).astype(jnp.int32)
    mesh = plsc.VectorSubcoreMesh(core_axis_name="c", subcore_axis_name="s")
    out = pl.kernel(
        _emb_body,
        out_type=jax.ShapeDtypeStruct((b, _DP), jnp.float32),
        mesh=mesh,
        compiler_params=pltpu.CompilerParams(use_tc_tiling_on_sc=False),
        scratch_types=[
            pltpu.VMEM((b // _NW,), jnp.int32),
            pltpu.VMEM((_CHUNK, _D), jnp.float32),
            pltpu.VMEM((_CHUNK, _D), jnp.float32),
            pltpu.SemaphoreType.DMA,
            pltpu.SemaphoreType.DMA,
            pltpu.SemaphoreType.DMA,
            pltpu.SemaphoreType.DMA,
        ],
    )(idx, weight)
    return out[:, :_D].reshape(s0, s1, _D)
